# baseline (speedup 1.0000x reference)
import functools

import jax
import jax.numpy as jnp
from jax import lax
from jax.experimental import pallas as pl
from jax.experimental.pallas import tpu as pltpu
from jax.experimental.pallas import tpu_sc as plsc

NUM_USERS = 1000000
NUM_ITEMS = 100000
EMBED_K = 16
BATCH = 16384

NC, NS, L = 2, 16, 16
NW = NC * NS
B_PER_W = BATCH // NW
CHUNK = 128
NCHUNK = B_PER_W // CHUNK
NGROUP = B_PER_W // L


def _sc_kernel(uidx_hbm, vidx_hbm, w_hbm, h_hbm, out_hbm,
               uidx_v, vidx_v, urows_v, vrows_v, out_v, sem):
    wid = lax.axis_index("s") * NC + lax.axis_index("c")
    base = wid * B_PER_W

    pltpu.sync_copy(uidx_hbm.at[pl.ds(wid * NCHUNK, NCHUNK)], uidx_v)
    pltpu.sync_copy(vidx_hbm.at[pl.ds(wid * NCHUNK, NCHUNK)], vidx_v)

    copies = []
    for j in range(NCHUNK):
        copies.append(pltpu.async_copy(
            w_hbm.at[uidx_v.at[j]], urows_v.at[pl.ds(j * CHUNK, CHUNK)], sem))
        copies.append(pltpu.async_copy(
            h_hbm.at[vidx_v.at[j]], vrows_v.at[pl.ds(j * CHUNK, CHUNK)], sem))
    for c in copies:
        c.wait()

    lane = lax.iota(jnp.int32, L)
    for g in range(NGROUP):
        rows = lane + (g * L)
        acc = jnp.zeros((L,), jnp.float32)
        for k in range(EMBED_K):
            col = jnp.full((L,), k, jnp.int32)
            u = plsc.load_gather(urows_v, [rows, col])
            v = plsc.load_gather(vrows_v, [rows, col])
            acc = acc + u * v
        out_v[pl.ds(g * L, L)] = 1.0 / (1.0 + jnp.exp(-acc))

    pltpu.sync_copy(out_v, out_hbm.at[pl.ds(base, B_PER_W)])


@jax.jit
def _mf_predict(uidx, vidx, W, H):
    mesh = plsc.VectorSubcoreMesh(
        core_axis_name="c", subcore_axis_name="s",
        num_cores=NC, num_subcores=NS)
    return pl.kernel(
        _sc_kernel,
        out_type=jax.ShapeDtypeStruct((BATCH,), jnp.float32),
        mesh=mesh,
        compiler_params=pltpu.CompilerParams(
            needs_layout_passes=False, use_tc_tiling_on_sc=False),
        scratch_types=[
            pltpu.VMEM((NCHUNK, CHUNK), jnp.int32),
            pltpu.VMEM((NCHUNK, CHUNK), jnp.int32),
            pltpu.VMEM((B_PER_W, EMBED_K), jnp.float32),
            pltpu.VMEM((B_PER_W, EMBED_K), jnp.float32),
            pltpu.VMEM((B_PER_W,), jnp.float32),
            pltpu.SemaphoreType.DMA,
        ],
    )(uidx, vidx, W, H)


def kernel(x, W, H):
    uidx = x[:, 0].astype(jnp.int32).reshape(BATCH // CHUNK, CHUNK)
    vidx = x[:, 1].astype(jnp.int32).reshape(BATCH // CHUNK, CHUNK)
    return _mf_predict(uidx, vidx, W, H)
